# baseline (speedup 1.0000x reference)
import jax
import jax.numpy as jnp
from jax import lax
from jax.experimental import pallas as pl
from jax.experimental.pallas import tpu as pltpu
from jax.experimental.pallas import tpu_sc as plsc

N = 10000
E = 160000
D = 128
NS = 16
LANES = 16
EPT = E // NS
B = 80
NCH = EPT // B
GPC = B // LANES
ACC_PAD = 10240
ROWS_PT = ACC_PAD // NS
DEN_PAD = 10240
BN = 1000
NBLK = N // BN


def _tc_proj_body(x_ref, w_ref, al_ref, ar_ref,
                  feat_ref, el_ref, er_ref, c_ref, sm):
    i = pl.program_id(1)
    feat = lax.dot_general(x_ref[...], w_ref[0], (((1,), (1,)), ((), ())),
                           preferred_element_type=jnp.float32)
    feat_ref[...] = feat
    el = jnp.sum(feat * al_ref[0], axis=1, keepdims=True)
    er = jnp.sum(feat * ar_ref[0], axis=1, keepdims=True)
    el_ref[...] = el
    er_ref[...] = er
    bl = jnp.max(el)
    br = jnp.max(er)

    @pl.when(i == 0)
    def _():
        sm[0] = bl
        sm[1] = br

    @pl.when(i > 0)
    def _():
        sm[0] = jnp.maximum(sm[0], bl)
        sm[1] = jnp.maximum(sm[1], br)

    @pl.when(i == NBLK - 1)
    def _():
        c_ref[...] = jnp.full((8, D), jnp.maximum(sm[0] + sm[1], 0.0),
                              jnp.float32)


_tc_proj = pl.pallas_call(
    _tc_proj_body,
    grid=(2, NBLK),
    in_specs=[
        pl.BlockSpec((BN, D), lambda e, i: (i, 0)),
        pl.BlockSpec((1, D, D), lambda e, i: (e, 0, 0)),
        pl.BlockSpec((1, 1, D), lambda e, i: (e, 0, 0)),
        pl.BlockSpec((1, 1, D), lambda e, i: (e, 0, 0)),
    ],
    out_specs=[
        pl.BlockSpec((BN, D), lambda e, i: (e * NBLK + i, 0)),
        pl.BlockSpec((BN, 1), lambda e, i: (e * NBLK + i, 0)),
        pl.BlockSpec((BN, 1), lambda e, i: (e * NBLK + i, 0)),
        pl.BlockSpec((8, D), lambda e, i: (e, 0)),
    ],
    out_shape=[
        jax.ShapeDtypeStruct((2 * N, D), jnp.float32),
        jax.ShapeDtypeStruct((2 * N, 1), jnp.float32),
        jax.ShapeDtypeStruct((2 * N, 1), jnp.float32),
        jax.ShapeDtypeStruct((16, D), jnp.float32),
    ],
    scratch_shapes=[pltpu.SMEM((2,), jnp.float32)],
)


def _sc_body(uvi_hbm, el_hbm, er_hbm, c_hbm, feat_hbm,
             h_hbm, den_hbm,
             uv2_c, ug2_c, vg2_c, p2_c, elu2_c, erv2_c, rows_v, c_v,
             acc_sh, den_sh, sem_e, sem_r, sem_d, sem_a):
    cid = lax.axis_index("c")
    sid = lax.axis_index("s")
    zero16 = jnp.zeros((LANES,), jnp.float32)

    def _zrow(r, _):
        for k in range(D // LANES):
            rows_v[0, r, pl.ds(k * LANES, LANES)] = zero16
        return 0
    lax.fori_loop(0, B, _zrow, 0)

    row0 = pl.multiple_of(sid * ROWS_PT, 8)
    for t in range(ROWS_PT // B):
        pltpu.sync_copy(rows_v.at[0], acc_sh.at[pl.ds(row0 + t * B, B)])
    for t in range(5):
        pltpu.sync_copy(rows_v.at[0, 0],
                        den_sh.at[pl.ds(sid * 640 + t * D, D)])

    pltpu.sync_copy(c_hbm.at[pl.ds(pl.multiple_of(cid * 8 * D, 8), LANES)],
                    c_v)
    cvec = c_v[...]
    plsc.subcore_barrier()

    jbase = cid * (E // B) + sid * NCH
    cofs = cid * N

    def _stage(j):
        b = lax.rem(j, 2)
        pltpu.sync_copy(uvi_hbm.at[jbase + j], uv2_c.at[b])

        def _idx(g, _2):
            o = pl.multiple_of(g * LANES, 8)
            ug2_c[b, pl.ds(o, LANES)] = uv2_c[b, 0, pl.ds(o, LANES)] + cofs
            vg2_c[b, pl.ds(o, LANES)] = uv2_c[b, 1, pl.ds(o, LANES)] + cofs
            return 0
        lax.fori_loop(0, GPC, _idx, 0)
        pltpu.async_copy(el_hbm.at[ug2_c.at[b]], elu2_c.at[b], sem_e)
        pltpu.async_copy(er_hbm.at[vg2_c.at[b]], erv2_c.at[b], sem_e)
        pltpu.async_copy(feat_hbm.at[ug2_c.at[b]], rows_v.at[b], sem_r)

    _stage(jnp.int32(0))

    def _chunk(j, _):
        b = lax.rem(j, 2)
        b1 = 1 - b
        pltpu.make_async_copy(el_hbm.at[ug2_c.at[b]], elu2_c.at[b],
                              sem_e).wait()
        pltpu.make_async_copy(er_hbm.at[vg2_c.at[b]], erv2_c.at[b],
                              sem_e).wait()

        def _grp(g, _2):
            o = pl.multiple_of(g * LANES, 8)
            s = elu2_c[b, pl.ds(o, LANES)] + erv2_c[b, pl.ds(o, LANES)]
            s = jnp.where(s >= 0.0, s, 0.2 * s)
            p2_c[b, pl.ds(o, LANES)] = jnp.exp(s - cvec)
            return 0
        lax.fori_loop(0, GPC, _grp, 0)
        pltpu.make_async_copy(feat_hbm.at[ug2_c.at[b]], rows_v.at[b],
                              sem_r).wait()

        @pl.when(j < NCH - 1)
        def _():
            @pl.when(j > 0)
            def _():
                pltpu.make_async_copy(p2_c.at[b1],
                                      den_sh.at[uv2_c.at[b1, 1]], sem_d).wait()
                pltpu.make_async_copy(rows_v.at[b1],
                                      acc_sh.at[uv2_c.at[b1, 1]], sem_a).wait()
            _stage(j + 1)

        pltpu.async_copy(p2_c.at[b], den_sh.at[uv2_c.at[b, 1]], sem_d, add=True)

        bb = jnp.full((LANES,), b, jnp.int32)
        one16 = jnp.full((LANES,), 1, jnp.int32)

        def _scale(r2, rr):
            r = 2 * r2
            af0 = plsc.load_gather(p2_c, [bb, rr])
            rr1 = rr + one16
            af1 = plsc.load_gather(p2_c, [bb, rr1])
            for k in range(D // LANES):
                sl = pl.ds(k * LANES, LANES)
                rows_v[b, r, sl] = rows_v[b, r, sl] * af0
                rows_v[b, r + 1, sl] = rows_v[b, r + 1, sl] * af1
            return rr1 + one16
        lax.fori_loop(0, B // 2, _scale, jnp.zeros((LANES,), jnp.int32))
        pltpu.async_copy(rows_v.at[b], acc_sh.at[uv2_c.at[b, 1]], sem_a,
                         add=True)
        return 0
    lax.fori_loop(0, NCH, _chunk, 0)

    for b in (0, 1):
        pltpu.make_async_copy(p2_c.at[b], den_sh.at[uv2_c.at[b, 1]],
                              sem_d).wait()
        pltpu.make_async_copy(rows_v.at[b], acc_sh.at[uv2_c.at[b, 1]],
                              sem_a).wait()
    plsc.subcore_barrier()

    pltpu.sync_copy(den_sh.at[pl.ds(sid * 640, 640)],
                    den_hbm.at[pl.ds(cid * DEN_PAD + sid * 640, 640)])

    @pl.when(sid < NS - 1)
    def _():
        hb = pl.multiple_of(cid * N + sid * ROWS_PT, 8)
        pltpu.sync_copy(acc_sh.at[pl.ds(row0, ROWS_PT)],
                        h_hbm.at[pl.ds(hb, ROWS_PT)])

    @pl.when(sid == NS - 1)
    def _():
        nrem = N - (NS - 1) * ROWS_PT
        hb = pl.multiple_of(cid * N + (NS - 1) * ROWS_PT, 8)
        pltpu.sync_copy(acc_sh.at[pl.ds(row0, nrem)],
                        h_hbm.at[pl.ds(hb, nrem)])


_sc_agg_built = None


def _sc_agg(*args):
    global _sc_agg_built
    if _sc_agg_built is None:
        _sc_agg_built = _build_sc_agg()
    return _sc_agg_built(*args)


def _build_sc_agg():
    return pl.kernel(
        _sc_body,
        out_type=(jax.ShapeDtypeStruct((2 * N, D), jnp.float32),
                  jax.ShapeDtypeStruct((2 * DEN_PAD,), jnp.float32)),
        mesh=plsc.VectorSubcoreMesh(core_axis_name="c", subcore_axis_name="s",
                                    num_cores=2, num_subcores=NS),
        compiler_params=pltpu.CompilerParams(needs_layout_passes=False),
        scratch_types=[
            pltpu.VMEM((2, 2, B), jnp.int32),
            pltpu.VMEM((2, B), jnp.int32),
            pltpu.VMEM((2, B), jnp.int32),
            pltpu.VMEM((2, B), jnp.float32),
            pltpu.VMEM((2, B), jnp.float32),
            pltpu.VMEM((2, B), jnp.float32),
            pltpu.VMEM((2, B, D), jnp.float32),
            pltpu.VMEM((LANES,), jnp.float32),
            pltpu.VMEM_SHARED((ACC_PAD, D), jnp.float32),
            pltpu.VMEM_SHARED((DEN_PAD,), jnp.float32),
            pltpu.SemaphoreType.DMA,
            pltpu.SemaphoreType.DMA,
            pltpu.SemaphoreType.DMA,
            pltpu.SemaphoreType.DMA,
        ],
    )


def _tc_wl_body(x_ref, h0_ref, h1_ref, d0_ref, d1d_ref, wx_ref, wd_ref,
                b_ref, o_ref):
    cdims = (((1,), (1,)), ((), ()))
    den0 = d0_ref[...]
    den1 = d1d_ref[...]
    h0 = h0_ref[...] / jnp.where(den0 == 0.0, 1.0, den0)
    h1 = h1_ref[...] / jnp.where(den1 == 0.0, 1.0, den1)
    t = lax.dot_general(x_ref[...], wx_ref[...], cdims,
                        preferred_element_type=jnp.float32)
    bias = b_ref[...]
    a1 = h0 + bias
    d1 = jnp.maximum(t + lax.dot_general(a1, wd_ref[...], cdims,
                                         preferred_element_type=jnp.float32),
                     0.0)
    a2 = d1 + h1 + bias
    o_ref[...] = jnp.maximum(
        t + lax.dot_general(a2, wd_ref[...], cdims,
                            preferred_element_type=jnp.float32), 0.0)


_tc_wl = pl.pallas_call(
    _tc_wl_body,
    grid=(NBLK,),
    in_specs=[
        pl.BlockSpec((BN, D), lambda i: (i, 0)),
        pl.BlockSpec((BN, D), lambda i: (i, 0)),
        pl.BlockSpec((BN, D), lambda i: (NBLK + i, 0)),
        pl.BlockSpec((BN, 1), lambda i: (i, 0)),
        pl.BlockSpec((BN, 1), lambda i: (i, 0)),
        pl.BlockSpec((D, D), lambda i: (0, 0)),
        pl.BlockSpec((D, D), lambda i: (0, 0)),
        pl.BlockSpec((1, D), lambda i: (0, 0)),
    ],
    out_specs=pl.BlockSpec((BN, D), lambda i: (i, 0)),
    out_shape=jax.ShapeDtypeStruct((N, D), jnp.float32),
)


def kernel(x, edge_index0, edge_index1, W0, attn_l0, attn_r0,
           W1, attn_l1, attn_r1, wl_W, bias):
    Wst = jnp.stack([W0, W1])
    ALst = jnp.stack([attn_l0, attn_l1]).reshape(2, 1, D)
    ARst = jnp.stack([attn_r0, attn_r1]).reshape(2, 1, D)
    FEAT, EL, ER, CC = _tc_proj(x, Wst, ALst, ARst)
    U = jnp.concatenate([edge_index0[0], edge_index1[0]])
    V = jnp.concatenate([edge_index0[1], edge_index1[1]])
    UVI = jnp.stack([U.reshape(2 * E // B, B), V.reshape(2 * E // B, B)],
                    axis=1)
    H, DEN = _sc_agg(UVI, EL.reshape(2 * N), ER.reshape(2 * N),
                     CC.reshape(16 * D), FEAT)
    den0 = DEN[:N].reshape(N, 1)
    den1 = DEN[DEN_PAD:DEN_PAD + N].reshape(N, 1)
    wlx = wl_W[:, :D]
    wld = wl_W[:, D:]
    return _tc_wl(x, H, H, den0, den1, wlx, wld, bias.reshape(1, D))


# speedup vs baseline: 1.1731x; 1.1731x over previous
import jax
import jax.numpy as jnp
from jax import lax
from jax.experimental import pallas as pl
from jax.experimental.pallas import tpu as pltpu
from jax.experimental.pallas import tpu_sc as plsc

N = 10000
E = 160000
D = 128
NS = 16
LANES = 16
EPT = E // NS
B = 80
NCH = EPT // B
GPC = B // LANES
ACC_PAD = 10240
ROWS_PT = ACC_PAD // NS
DEN_PAD = 10240
BN = 1000
NBLK = N // BN


def _tc_proj_body(x_ref, w_ref, al_ref, ar_ref,
                  feat_ref, el_ref, er_ref, c_ref, sm):
    i = pl.program_id(1)
    feat = lax.dot_general(x_ref[...], w_ref[0], (((1,), (1,)), ((), ())),
                           preferred_element_type=jnp.float32)
    feat_ref[...] = feat
    el = jnp.sum(feat * al_ref[0], axis=1, keepdims=True)
    er = jnp.sum(feat * ar_ref[0], axis=1, keepdims=True)
    el_ref[...] = el
    er_ref[...] = er
    bl = jnp.max(el)
    br = jnp.max(er)

    @pl.when(i == 0)
    def _():
        sm[0] = bl
        sm[1] = br

    @pl.when(i > 0)
    def _():
        sm[0] = jnp.maximum(sm[0], bl)
        sm[1] = jnp.maximum(sm[1], br)

    @pl.when(i == NBLK - 1)
    def _():
        c_ref[...] = jnp.full((8, D), jnp.maximum(sm[0] + sm[1], 0.0),
                              jnp.float32)


_tc_proj = pl.pallas_call(
    _tc_proj_body,
    grid=(2, NBLK),
    in_specs=[
        pl.BlockSpec((BN, D), lambda e, i: (i, 0)),
        pl.BlockSpec((1, D, D), lambda e, i: (e, 0, 0)),
        pl.BlockSpec((1, 1, D), lambda e, i: (e, 0, 0)),
        pl.BlockSpec((1, 1, D), lambda e, i: (e, 0, 0)),
    ],
    out_specs=[
        pl.BlockSpec((BN, D), lambda e, i: (e * NBLK + i, 0)),
        pl.BlockSpec((BN, 1), lambda e, i: (e * NBLK + i, 0)),
        pl.BlockSpec((BN, 1), lambda e, i: (e * NBLK + i, 0)),
        pl.BlockSpec((8, D), lambda e, i: (e, 0)),
    ],
    out_shape=[
        jax.ShapeDtypeStruct((2 * N, D), jnp.float32),
        jax.ShapeDtypeStruct((2 * N, 1), jnp.float32),
        jax.ShapeDtypeStruct((2 * N, 1), jnp.float32),
        jax.ShapeDtypeStruct((16, D), jnp.float32),
    ],
    scratch_shapes=[pltpu.SMEM((2,), jnp.float32)],
)


def _sc_body(u_hbm, v_hbm, el_hbm, er_hbm, c_hbm, feat_hbm,
             h_hbm, den_hbm,
             uv2_c, ug2_c, vg2_c, p2_c, elu2_c, erv2_c, rows_v, c_v,
             acc_sh, den_sh, elr_sh, sem_e, sem_r, sem_d, sem_a):
    cid = lax.axis_index("c")
    sid = lax.axis_index("s")
    zero16 = jnp.zeros((LANES,), jnp.float32)

    def _zrow(r, _):
        for k in range(D // LANES):
            rows_v[0, r, pl.ds(k * LANES, LANES)] = zero16
        return 0
    lax.fori_loop(0, B, _zrow, 0)

    row0 = pl.multiple_of(sid * ROWS_PT, 8)
    for t in range(ROWS_PT // B):
        pltpu.sync_copy(rows_v.at[0], acc_sh.at[pl.ds(row0 + t * B, B)])
    for t in range(5):
        pltpu.sync_copy(rows_v.at[0, 0],
                        den_sh.at[pl.ds(sid * 640 + t * D, D)])

    nb = pl.multiple_of(cid * N, 8)

    def _bounce(src_hbm, dst_base, t):
        so = pl.multiple_of(sid * 640 + t * B, 8)
        pltpu.sync_copy(src_hbm.at[pl.ds(nb + so, B)], elu2_c.at[0])
        pltpu.sync_copy(elu2_c.at[0], elr_sh.at[pl.ds(dst_base + so, B)])

    @pl.when(sid < NS - 1)
    def _():
        for t in range(8):
            _bounce(el_hbm, 0, t)
            _bounce(er_hbm, DEN_PAD, t)

    @pl.when(sid == NS - 1)
    def _():
        for t in range(5):
            _bounce(el_hbm, 0, t)
            _bounce(er_hbm, DEN_PAD, t)

    pltpu.sync_copy(c_hbm.at[pl.ds(pl.multiple_of(cid * 8 * D, 8), LANES)],
                    c_v)
    cvec = c_v[...]
    plsc.subcore_barrier()

    ebase = cid * E + sid * EPT
    cofs = cid * N

    def _stage(j):
        b = lax.rem(j, 2)
        off = pl.multiple_of(ebase + j * B, 8)
        pltpu.sync_copy(u_hbm.at[pl.ds(off, B)], uv2_c.at[b, 0])
        pltpu.sync_copy(v_hbm.at[pl.ds(off, B)], uv2_c.at[b, 1])

        def _idx(g, _2):
            o = pl.multiple_of(g * LANES, 8)
            ug2_c[b, pl.ds(o, LANES)] = uv2_c[b, 0, pl.ds(o, LANES)] + cofs
            vg2_c[b, pl.ds(o, LANES)] = uv2_c[b, 1, pl.ds(o, LANES)] + DEN_PAD
            return 0
        lax.fori_loop(0, GPC, _idx, 0)
        pltpu.async_copy(elr_sh.at[uv2_c.at[b, 0]], elu2_c.at[b], sem_e)
        pltpu.async_copy(elr_sh.at[vg2_c.at[b]], erv2_c.at[b], sem_e)
        pltpu.async_copy(feat_hbm.at[ug2_c.at[b]], rows_v.at[b], sem_r)

    _stage(jnp.int32(0))

    def _chunk(j, _):
        b = lax.rem(j, 2)
        b1 = 1 - b
        pltpu.make_async_copy(elr_sh.at[uv2_c.at[b, 0]], elu2_c.at[b],
                              sem_e).wait()
        pltpu.make_async_copy(elr_sh.at[vg2_c.at[b]], erv2_c.at[b],
                              sem_e).wait()

        def _grp(g, _2):
            o = pl.multiple_of(g * LANES, 8)
            s = elu2_c[b, pl.ds(o, LANES)] + erv2_c[b, pl.ds(o, LANES)]
            s = jnp.where(s >= 0.0, s, 0.2 * s)
            p2_c[b, pl.ds(o, LANES)] = jnp.exp(s - cvec)
            return 0
        lax.fori_loop(0, GPC, _grp, 0)
        pltpu.make_async_copy(feat_hbm.at[ug2_c.at[b]], rows_v.at[b],
                              sem_r).wait()

        @pl.when(j < NCH - 1)
        def _():
            @pl.when(j > 0)
            def _():
                pltpu.make_async_copy(p2_c.at[b1],
                                      den_sh.at[uv2_c.at[b1, 1]], sem_d).wait()
                pltpu.make_async_copy(rows_v.at[b1],
                                      acc_sh.at[uv2_c.at[b1, 1]], sem_a).wait()
            _stage(j + 1)

        pltpu.async_copy(p2_c.at[b], den_sh.at[uv2_c.at[b, 1]], sem_d, add=True)

        bb = jnp.full((LANES,), b, jnp.int32)

        def _scale(r, _2):
            rr = jnp.full((LANES,), r, jnp.int32)
            af = plsc.load_gather(p2_c, [bb, rr])
            for k in range(D // LANES):
                sl = pl.ds(k * LANES, LANES)
                rows_v[b, r, sl] = rows_v[b, r, sl] * af
            return 0
        lax.fori_loop(0, B, _scale, 0)
        pltpu.async_copy(rows_v.at[b], acc_sh.at[uv2_c.at[b, 1]], sem_a,
                         add=True)
        return 0
    lax.fori_loop(0, NCH, _chunk, 0)

    for b in (0, 1):
        pltpu.make_async_copy(p2_c.at[b], den_sh.at[uv2_c.at[b, 1]],
                              sem_d).wait()
        pltpu.make_async_copy(rows_v.at[b], acc_sh.at[uv2_c.at[b, 1]],
                              sem_a).wait()
    plsc.subcore_barrier()

    pltpu.sync_copy(den_sh.at[pl.ds(sid * 640, 640)],
                    den_hbm.at[pl.ds(cid * DEN_PAD + sid * 640, 640)])

    @pl.when(sid < NS - 1)
    def _():
        hb = pl.multiple_of(cid * N + sid * ROWS_PT, 8)
        pltpu.sync_copy(acc_sh.at[pl.ds(row0, ROWS_PT)],
                        h_hbm.at[pl.ds(hb, ROWS_PT)])

    @pl.when(sid == NS - 1)
    def _():
        nrem = N - (NS - 1) * ROWS_PT
        hb = pl.multiple_of(cid * N + (NS - 1) * ROWS_PT, 8)
        pltpu.sync_copy(acc_sh.at[pl.ds(row0, nrem)],
                        h_hbm.at[pl.ds(hb, nrem)])


_sc_agg_built = None


def _sc_agg(*args):
    global _sc_agg_built
    if _sc_agg_built is None:
        _sc_agg_built = _build_sc_agg()
    return _sc_agg_built(*args)


def _build_sc_agg():
    return pl.kernel(
        _sc_body,
        out_type=(jax.ShapeDtypeStruct((2 * N, D), jnp.float32),
                  jax.ShapeDtypeStruct((2 * DEN_PAD,), jnp.float32)),
        mesh=plsc.VectorSubcoreMesh(core_axis_name="c", subcore_axis_name="s",
                                    num_cores=2, num_subcores=NS),
        compiler_params=pltpu.CompilerParams(needs_layout_passes=False),
        scratch_types=[
            pltpu.VMEM((2, 2, B), jnp.int32),
            pltpu.VMEM((2, B), jnp.int32),
            pltpu.VMEM((2, B), jnp.int32),
            pltpu.VMEM((2, B), jnp.float32),
            pltpu.VMEM((2, B), jnp.float32),
            pltpu.VMEM((2, B), jnp.float32),
            pltpu.VMEM((2, B, D), jnp.float32),
            pltpu.VMEM((LANES,), jnp.float32),
            pltpu.VMEM_SHARED((ACC_PAD, D), jnp.float32),
            pltpu.VMEM_SHARED((DEN_PAD,), jnp.float32),
            pltpu.VMEM_SHARED((2 * DEN_PAD,), jnp.float32),
            pltpu.SemaphoreType.DMA,
            pltpu.SemaphoreType.DMA,
            pltpu.SemaphoreType.DMA,
            pltpu.SemaphoreType.DMA,
        ],
    )


def _tc_wl_body(x_ref, h0_ref, h1_ref, d0_ref, d1d_ref, wx_ref, wd_ref,
                b_ref, o_ref):
    cdims = (((1,), (1,)), ((), ()))
    den0 = d0_ref[...]
    den1 = d1d_ref[...]
    h0 = h0_ref[...] / jnp.where(den0 == 0.0, 1.0, den0)
    h1 = h1_ref[...] / jnp.where(den1 == 0.0, 1.0, den1)
    t = lax.dot_general(x_ref[...], wx_ref[...], cdims,
                        preferred_element_type=jnp.float32)
    bias = b_ref[...]
    a1 = h0 + bias
    d1 = jnp.maximum(t + lax.dot_general(a1, wd_ref[...], cdims,
                                         preferred_element_type=jnp.float32),
                     0.0)
    a2 = d1 + h1 + bias
    o_ref[...] = jnp.maximum(
        t + lax.dot_general(a2, wd_ref[...], cdims,
                            preferred_element_type=jnp.float32), 0.0)


_tc_wl = pl.pallas_call(
    _tc_wl_body,
    grid=(NBLK,),
    in_specs=[
        pl.BlockSpec((BN, D), lambda i: (i, 0)),
        pl.BlockSpec((BN, D), lambda i: (i, 0)),
        pl.BlockSpec((BN, D), lambda i: (NBLK + i, 0)),
        pl.BlockSpec((BN, 1), lambda i: (i, 0)),
        pl.BlockSpec((BN, 1), lambda i: (i, 0)),
        pl.BlockSpec((D, D), lambda i: (0, 0)),
        pl.BlockSpec((D, D), lambda i: (0, 0)),
        pl.BlockSpec((1, D), lambda i: (0, 0)),
    ],
    out_specs=pl.BlockSpec((BN, D), lambda i: (i, 0)),
    out_shape=jax.ShapeDtypeStruct((N, D), jnp.float32),
)


def kernel(x, edge_index0, edge_index1, W0, attn_l0, attn_r0,
           W1, attn_l1, attn_r1, wl_W, bias):
    Wst = jnp.stack([W0, W1])
    ALst = jnp.stack([attn_l0, attn_l1]).reshape(2, 1, D)
    ARst = jnp.stack([attn_r0, attn_r1]).reshape(2, 1, D)
    FEAT, EL, ER, CC = _tc_proj(x, Wst, ALst, ARst)
    U = jnp.concatenate([edge_index0[0], edge_index1[0]])
    V = jnp.concatenate([edge_index0[1], edge_index1[1]])
    H, DEN = _sc_agg(U, V, EL.reshape(2 * N), ER.reshape(2 * N),
                     CC.reshape(16 * D), FEAT)
    den0 = DEN[:N].reshape(N, 1)
    den1 = DEN[DEN_PAD:DEN_PAD + N].reshape(N, 1)
    wlx = wl_W[:, :D]
    wld = wl_W[:, D:]
    return _tc_wl(x, H, H, den0, den1, wlx, wld, bias.reshape(1, D))


# speedup vs baseline: 1.2069x; 1.0288x over previous
import jax
import jax.numpy as jnp
from jax import lax
from jax.experimental import pallas as pl
from jax.experimental.pallas import tpu as pltpu
from jax.experimental.pallas import tpu_sc as plsc

N = 10000
E = 160000
D = 128
NS = 16
LANES = 16
EPT = E // NS
B = 80
NCH = EPT // B
GPC = B // LANES
ACC_PAD = 10240
ROWS_PT = ACC_PAD // NS
DEN_PAD = 10240
BN = 1000
NBLK = N // BN


def _tc_proj_body(x_ref, w_ref, al_ref, ar_ref,
                  feat_ref, el_ref, er_ref, c_ref, sm):
    i = pl.program_id(1)
    feat = lax.dot_general(x_ref[...], w_ref[0], (((1,), (1,)), ((), ())),
                           preferred_element_type=jnp.float32)
    feat_ref[...] = feat
    el = jnp.sum(feat * al_ref[0], axis=1, keepdims=True)
    er = jnp.sum(feat * ar_ref[0], axis=1, keepdims=True)
    el_ref[...] = el
    er_ref[...] = er
    bl = jnp.max(el)
    br = jnp.max(er)

    @pl.when(i == 0)
    def _():
        sm[0] = bl
        sm[1] = br

    @pl.when(i > 0)
    def _():
        sm[0] = jnp.maximum(sm[0], bl)
        sm[1] = jnp.maximum(sm[1], br)

    @pl.when(i == NBLK - 1)
    def _():
        c_ref[...] = jnp.full((8, D), jnp.maximum(sm[0] + sm[1], 0.0),
                              jnp.float32)


_tc_proj = pl.pallas_call(
    _tc_proj_body,
    grid=(2, NBLK),
    in_specs=[
        pl.BlockSpec((BN, D), lambda e, i: (i, 0)),
        pl.BlockSpec((1, D, D), lambda e, i: (e, 0, 0)),
        pl.BlockSpec((1, 1, D), lambda e, i: (e, 0, 0)),
        pl.BlockSpec((1, 1, D), lambda e, i: (e, 0, 0)),
    ],
    out_specs=[
        pl.BlockSpec((BN, D), lambda e, i: (e * NBLK + i, 0)),
        pl.BlockSpec((BN, 1), lambda e, i: (e * NBLK + i, 0)),
        pl.BlockSpec((BN, 1), lambda e, i: (e * NBLK + i, 0)),
        pl.BlockSpec((8, D), lambda e, i: (e, 0)),
    ],
    out_shape=[
        jax.ShapeDtypeStruct((2 * N, D), jnp.float32),
        jax.ShapeDtypeStruct((2 * N, 1), jnp.float32),
        jax.ShapeDtypeStruct((2 * N, 1), jnp.float32),
        jax.ShapeDtypeStruct((16, D), jnp.float32),
    ],
    scratch_shapes=[pltpu.SMEM((2,), jnp.float32)],
)


def _sc_body(u_hbm, v_hbm, el_hbm, er_hbm, c_hbm, feat_hbm,
             h_hbm, den_hbm,
             uv2_c, ug2_c, vg2_c, p2_c, elu2_c, erv2_c, rows_v, c_v,
             acc_sh, den_sh, sem_e, sem_e1, sem_r, sem_r1,
             sem_d, sem_d1, sem_a, sem_a1):
    cid = lax.axis_index("c")
    sid = lax.axis_index("s")
    zero16 = jnp.zeros((LANES,), jnp.float32)

    def _zrow(r, _):
        for k in range(D // LANES):
            rows_v[0, r, pl.ds(k * LANES, LANES)] = zero16
        return 0
    lax.fori_loop(0, B, _zrow, 0)

    row0 = pl.multiple_of(sid * ROWS_PT, 8)
    for t in range(ROWS_PT // B):
        pltpu.sync_copy(rows_v.at[0], acc_sh.at[pl.ds(row0 + t * B, B)])
    for t in range(5):
        pltpu.sync_copy(rows_v.at[0, 0],
                        den_sh.at[pl.ds(sid * 640 + t * D, D)])

    pltpu.sync_copy(c_hbm.at[pl.ds(pl.multiple_of(cid * 8 * D, 8), LANES)],
                    c_v)
    cvec = c_v[...]
    plsc.subcore_barrier()

    ebase = cid * E + sid * EPT
    cofs = cid * N

    sem_es = (sem_e, sem_e1)
    sem_rs = (sem_r, sem_r1)
    sem_ds = (sem_d, sem_d1)
    sem_as = (sem_a, sem_a1)

    def _stage(j, s):
        off = pl.multiple_of(ebase + j * B, 8)
        pltpu.sync_copy(u_hbm.at[pl.ds(off, B)], uv2_c.at[s, 0])
        pltpu.sync_copy(v_hbm.at[pl.ds(off, B)], uv2_c.at[s, 1])

        def _idx(g, _2):
            o = pl.multiple_of(g * LANES, 8)
            ug2_c[s, pl.ds(o, LANES)] = uv2_c[s, 0, pl.ds(o, LANES)] + cofs
            vg2_c[s, pl.ds(o, LANES)] = uv2_c[s, 1, pl.ds(o, LANES)] + cofs
            return 0
        lax.fori_loop(0, GPC, _idx, 0)
        pltpu.async_copy(el_hbm.at[ug2_c.at[s]], elu2_c.at[s], sem_es[s])
        pltpu.async_copy(er_hbm.at[vg2_c.at[s]], erv2_c.at[s], sem_es[s])
        pltpu.async_copy(feat_hbm.at[ug2_c.at[s]], rows_v.at[s], sem_rs[s])

    def _drain(s1):
        pltpu.make_async_copy(p2_c.at[s1], den_sh.at[uv2_c.at[s1, 1]],
                              sem_ds[s1]).wait()
        pltpu.make_async_copy(rows_v.at[s1], acc_sh.at[uv2_c.at[s1, 1]],
                              sem_as[s1]).wait()

    def _iter(j, s, drain_pred, stage_next):
        s1 = 1 - s
        pltpu.make_async_copy(el_hbm.at[ug2_c.at[s]], elu2_c.at[s],
                              sem_es[s]).wait()
        pltpu.make_async_copy(er_hbm.at[vg2_c.at[s]], erv2_c.at[s],
                              sem_es[s]).wait()

        def _grp(g, _2):
            o = pl.multiple_of(g * LANES, 8)
            z = elu2_c[s, pl.ds(o, LANES)] + erv2_c[s, pl.ds(o, LANES)]
            z = jnp.where(z >= 0.0, z, 0.2 * z)
            p2_c[s, pl.ds(o, LANES)] = jnp.exp(z - cvec)
            return 0
        lax.fori_loop(0, GPC, _grp, 0)

        if drain_pred is None:
            _drain(s1)
        else:
            pl.when(drain_pred)(lambda: _drain(s1))
        pltpu.async_copy(p2_c.at[s], den_sh.at[uv2_c.at[s, 1]],
                         sem_ds[s], add=True)
        if stage_next:
            _stage(j + 1, s1)
        pltpu.make_async_copy(feat_hbm.at[ug2_c.at[s]], rows_v.at[s],
                              sem_rs[s]).wait()

        def _scale(r, _2):
            rr = jnp.full((LANES,), r, jnp.int32)
            af = plsc.load_gather(p2_c.at[s], [rr])
            for k in range(D // LANES):
                sl = pl.ds(k * LANES, LANES)
                rows_v[s, r, sl] = rows_v[s, r, sl] * af
            return 0
        lax.fori_loop(0, B, _scale, 0)
        pltpu.async_copy(rows_v.at[s], acc_sh.at[uv2_c.at[s, 1]],
                         sem_as[s], add=True)

    _stage(0, 0)

    def _pair(t, _):
        j0 = 2 * t
        _iter(j0, 0, t > 0, True)
        _iter(j0 + 1, 1, None, True)
        return 0
    lax.fori_loop(0, NCH // 2, _pair, 0)
    _iter(NCH - 1, 0, None, False)
    _drain(0)
    plsc.subcore_barrier()

    pltpu.sync_copy(den_sh.at[pl.ds(sid * 640, 640)],
                    den_hbm.at[pl.ds(cid * DEN_PAD + sid * 640, 640)])

    @pl.when(sid < NS - 1)
    def _():
        hb = pl.multiple_of(cid * N + sid * ROWS_PT, 8)
        pltpu.sync_copy(acc_sh.at[pl.ds(row0, ROWS_PT)],
                        h_hbm.at[pl.ds(hb, ROWS_PT)])

    @pl.when(sid == NS - 1)
    def _():
        nrem = N - (NS - 1) * ROWS_PT
        hb = pl.multiple_of(cid * N + (NS - 1) * ROWS_PT, 8)
        pltpu.sync_copy(acc_sh.at[pl.ds(row0, nrem)],
                        h_hbm.at[pl.ds(hb, nrem)])


_sc_agg_built = None


def _sc_agg(*args):
    global _sc_agg_built
    if _sc_agg_built is None:
        _sc_agg_built = _build_sc_agg()
    return _sc_agg_built(*args)


def _build_sc_agg():
    return pl.kernel(
        _sc_body,
        out_type=(jax.ShapeDtypeStruct((2 * N, D), jnp.float32),
                  jax.ShapeDtypeStruct((2 * DEN_PAD,), jnp.float32)),
        mesh=plsc.VectorSubcoreMesh(core_axis_name="c", subcore_axis_name="s",
                                    num_cores=2, num_subcores=NS),
        compiler_params=pltpu.CompilerParams(needs_layout_passes=False),
        scratch_types=[
            pltpu.VMEM((2, 2, B), jnp.int32),
            pltpu.VMEM((2, B), jnp.int32),
            pltpu.VMEM((2, B), jnp.int32),
            pltpu.VMEM((2, B), jnp.float32),
            pltpu.VMEM((2, B), jnp.float32),
            pltpu.VMEM((2, B), jnp.float32),
            pltpu.VMEM((2, B, D), jnp.float32),
            pltpu.VMEM((LANES,), jnp.float32),
            pltpu.VMEM_SHARED((ACC_PAD, D), jnp.float32),
            pltpu.VMEM_SHARED((DEN_PAD,), jnp.float32),
            pltpu.SemaphoreType.DMA,
            pltpu.SemaphoreType.DMA,
            pltpu.SemaphoreType.DMA,
            pltpu.SemaphoreType.DMA,
            pltpu.SemaphoreType.DMA,
            pltpu.SemaphoreType.DMA,
            pltpu.SemaphoreType.DMA,
            pltpu.SemaphoreType.DMA,
        ],
    )


def _tc_wl_body(x_ref, h0_ref, h1_ref, d0_ref, d1d_ref, wx_ref, wd_ref,
                b_ref, o_ref):
    cdims = (((1,), (1,)), ((), ()))
    den0 = d0_ref[...]
    den1 = d1d_ref[...]
    h0 = h0_ref[...] / jnp.where(den0 == 0.0, 1.0, den0)
    h1 = h1_ref[...] / jnp.where(den1 == 0.0, 1.0, den1)
    t = lax.dot_general(x_ref[...], wx_ref[...], cdims,
                        preferred_element_type=jnp.float32)
    bias = b_ref[...]
    a1 = h0 + bias
    d1 = jnp.maximum(t + lax.dot_general(a1, wd_ref[...], cdims,
                                         preferred_element_type=jnp.float32),
                     0.0)
    a2 = d1 + h1 + bias
    o_ref[...] = jnp.maximum(
        t + lax.dot_general(a2, wd_ref[...], cdims,
                            preferred_element_type=jnp.float32), 0.0)


_tc_wl = pl.pallas_call(
    _tc_wl_body,
    grid=(NBLK,),
    in_specs=[
        pl.BlockSpec((BN, D), lambda i: (i, 0)),
        pl.BlockSpec((BN, D), lambda i: (i, 0)),
        pl.BlockSpec((BN, D), lambda i: (NBLK + i, 0)),
        pl.BlockSpec((BN, 1), lambda i: (i, 0)),
        pl.BlockSpec((BN, 1), lambda i: (i, 0)),
        pl.BlockSpec((D, D), lambda i: (0, 0)),
        pl.BlockSpec((D, D), lambda i: (0, 0)),
        pl.BlockSpec((1, D), lambda i: (0, 0)),
    ],
    out_specs=pl.BlockSpec((BN, D), lambda i: (i, 0)),
    out_shape=jax.ShapeDtypeStruct((N, D), jnp.float32),
)


def kernel(x, edge_index0, edge_index1, W0, attn_l0, attn_r0,
           W1, attn_l1, attn_r1, wl_W, bias):
    Wst = jnp.stack([W0, W1])
    ALst = jnp.stack([attn_l0, attn_l1]).reshape(2, 1, D)
    ARst = jnp.stack([attn_r0, attn_r1]).reshape(2, 1, D)
    FEAT, EL, ER, CC = _tc_proj(x, Wst, ALst, ARst)
    U = jnp.concatenate([edge_index0[0], edge_index1[0]])
    V = jnp.concatenate([edge_index0[1], edge_index1[1]])
    H, DEN = _sc_agg(U, V, EL.reshape(2 * N), ER.reshape(2 * N),
                     CC.reshape(16 * D), FEAT)
    den0 = DEN[:N].reshape(N, 1)
    den1 = DEN[DEN_PAD:DEN_PAD + N].reshape(N, 1)
    wlx = wl_W[:, :D]
    wld = wl_W[:, D:]
    return _tc_wl(x, H, H, den0, den1, wlx, wld, bias.reshape(1, D))
